# baseline (speedup 1.0000x reference)
import functools

import jax
import jax.numpy as jnp
from jax.experimental import pallas as pl
from jax.experimental.pallas import tpu as pltpu

_C_IN = 32
_C_OUT = 192
_BN_EPS = 1e-5
_BLOCK_BATCH = 2


def _fused_kernel(x71_ref, x57_ref, w_ref, gamma_ref, beta_ref, o_hbm,
                  xs_scr, s_scr, g_scr, wf_scr, bf_scr, obuf, osem,
                  *, count, bb, steps, nbuf):
    phase = pl.program_id(0)
    step = pl.program_id(1)

    @pl.when(jnp.logical_and(phase == 0, step == 0))
    def _init():
        s_scr[...] = jnp.zeros_like(s_scr)
        g_scr[...] = jnp.zeros_like(g_scr)

    @pl.when(phase == 0)
    def _stats():
        x = x71_ref[...] + x57_ref[...]
        xs_scr[pl.ds(step * bb, bb)] = x.astype(jnp.bfloat16)
        s_scr[...] += jnp.sum(x, axis=(0, 2))[:, None]
        g = jnp.zeros((_C_IN, _C_IN), jnp.float32)
        for b in range(bb):
            g = g + jax.lax.dot_general(
                x[b], x[b], (((1,), (1,)), ((), ())),
                preferred_element_type=jnp.float32)
        g_scr[...] += g

    @pl.when(jnp.logical_and(phase == 1, step == 0))
    def _fold():
        inv = 1.0 / count
        mean_x = s_scr[...] * inv
        exx = g_scr[...] * inv
        w = w_ref[...]
        mean_y = jnp.dot(w, mean_x, preferred_element_type=jnp.float32)
        e_y2 = jnp.sum(jnp.dot(w, exx, preferred_element_type=jnp.float32) * w,
                       axis=1, keepdims=True)
        var_y = jnp.maximum(e_y2 - mean_y * mean_y, 0.0)
        scale = gamma_ref[...] * jax.lax.rsqrt(var_y + _BN_EPS)
        wf_scr[...] = (w * scale).astype(jnp.bfloat16)
        bf_scr[...] = beta_ref[...] - mean_y * scale

    @pl.when(phase == 1)
    def _conv():
        slot = jax.lax.rem(step, nbuf)

        @pl.when(step >= nbuf)
        def _reuse_wait():
            pltpu.make_async_copy(obuf.at[slot], o_hbm.at[pl.ds(0, bb)],
                                  osem.at[slot]).wait()

        wf = wf_scr[...]
        bias = bf_scr[...]
        xsb = xs_scr[pl.ds(step * bb, bb)]
        for b in range(bb):
            y = jnp.dot(wf, xsb[b],
                        preferred_element_type=jnp.float32)
            obuf[slot, b] = y + bias

        pltpu.make_async_copy(obuf.at[slot], o_hbm.at[pl.ds(step * bb, bb)],
                              osem.at[slot]).start()

        @pl.when(step == steps - 1)
        def _drain():
            for k in range(min(nbuf, steps)):
                pltpu.make_async_copy(obuf.at[k], o_hbm.at[pl.ds(0, bb)],
                                      osem.at[k]).wait()


def kernel(x71, x57, weight, gamma, beta):
    n, c, h, w = x71.shape
    assert c == _C_IN and x57.shape == x71.shape
    hw = h * w
    bb = _BLOCK_BATCH if n % _BLOCK_BATCH == 0 else 1
    steps = n // bb
    nbuf = min(2, steps)

    x71_r = x71.reshape(n, _C_IN, hw)
    x57_r = x57.reshape(n, _C_IN, hw)
    w_mat = weight.astype(jnp.float32).reshape(_C_OUT, _C_IN)
    g_col = gamma.astype(jnp.float32).reshape(_C_OUT, 1)
    b_col = beta.astype(jnp.float32).reshape(_C_OUT, 1)

    in_map = lambda ph, st: (st * (1 - ph), 0, 0)

    out = pl.pallas_call(
        functools.partial(_fused_kernel, count=float(n * hw), bb=bb,
                          steps=steps, nbuf=nbuf),
        out_shape=jax.ShapeDtypeStruct((n, _C_OUT, hw), jnp.float32),
        grid=(2, steps),
        in_specs=[
            pl.BlockSpec((bb, _C_IN, hw), in_map),
            pl.BlockSpec((bb, _C_IN, hw), in_map),
            pl.BlockSpec((_C_OUT, _C_IN), lambda ph, st: (0, 0)),
            pl.BlockSpec((_C_OUT, 1), lambda ph, st: (0, 0)),
            pl.BlockSpec((_C_OUT, 1), lambda ph, st: (0, 0)),
        ],
        out_specs=pl.BlockSpec(memory_space=pl.ANY),
        scratch_shapes=[
            pltpu.VMEM((n, _C_IN, hw), jnp.bfloat16),
            pltpu.VMEM((_C_IN, 1), jnp.float32),
            pltpu.VMEM((_C_IN, _C_IN), jnp.float32),
            pltpu.VMEM((_C_OUT, _C_IN), jnp.bfloat16),
            pltpu.VMEM((_C_OUT, 1), jnp.float32),
            pltpu.VMEM((2, bb, _C_OUT, hw), jnp.float32),
            pltpu.SemaphoreType.DMA((2,)),
        ],
        compiler_params=pltpu.CompilerParams(
            dimension_semantics=("arbitrary", "arbitrary")),
    )(x71_r, x57_r, w_mat, g_col, b_col)

    return out.reshape(n, _C_OUT, h, w)


# speedup vs baseline: 1.0038x; 1.0038x over previous
import functools

import jax
import jax.numpy as jnp
from jax.experimental import pallas as pl
from jax.experimental.pallas import tpu as pltpu

_C_IN = 32
_C_OUT = 192
_BN_EPS = 1e-5
_BLOCK_BATCH = 2


def _fused_kernel(x71_ref, x57_ref, w_ref, gamma_ref, beta_ref, o_ref,
                  xs_scr, s_scr, g_scr, wf_scr, bf_scr, *, count, bb, steps):
    phase = pl.program_id(0)
    step = pl.program_id(1)

    @pl.when(jnp.logical_and(phase == 0, step == 0))
    def _init():
        s_scr[...] = jnp.zeros_like(s_scr)
        g_scr[...] = jnp.zeros_like(g_scr)

    @pl.when(phase == 0)
    def _stats():
        x = x71_ref[...] + x57_ref[...]
        xs_scr[pl.ds(step * bb, bb)] = x.astype(jnp.bfloat16)
        s_scr[...] += jnp.sum(x, axis=(0, 2))[:, None]
        g = jnp.zeros((_C_IN, _C_IN), jnp.float32)
        for b in range(bb):
            g = g + jax.lax.dot_general(
                x[b], x[b], (((1,), (1,)), ((), ())),
                preferred_element_type=jnp.float32)
        g_scr[...] += g

    @pl.when(jnp.logical_and(phase == 1, step == 0))
    def _fold():
        inv = 1.0 / count
        mean_x = s_scr[...] * inv
        exx = g_scr[...] * inv
        w = w_ref[...]
        mean_y = jnp.dot(w, mean_x, preferred_element_type=jnp.float32)
        e_y2 = jnp.sum(jnp.dot(w, exx, preferred_element_type=jnp.float32) * w,
                       axis=1, keepdims=True)
        var_y = jnp.maximum(e_y2 - mean_y * mean_y, 0.0)
        scale = gamma_ref[...] * jax.lax.rsqrt(var_y + _BN_EPS)
        wf_scr[...] = (w * scale).astype(jnp.bfloat16)
        bf_scr[...] = beta_ref[...] - mean_y * scale

    @pl.when(phase == 1)
    def _conv():
        wf = wf_scr[...]
        bias = bf_scr[...]
        xsb = xs_scr[pl.ds(step * bb, bb)]
        for b in range(bb):
            y = jnp.dot(wf, xsb[b],
                        preferred_element_type=jnp.float32)
            o_ref[b] = y + bias


def kernel(x71, x57, weight, gamma, beta):
    n, c, h, w = x71.shape
    assert c == _C_IN and x57.shape == x71.shape
    hw = h * w
    bb = _BLOCK_BATCH if n % _BLOCK_BATCH == 0 else 1
    steps = n // bb

    x71_r = x71.reshape(n, _C_IN, hw)
    x57_r = x57.reshape(n, _C_IN, hw)
    w_mat = weight.astype(jnp.float32).reshape(_C_OUT, _C_IN)
    g_col = gamma.astype(jnp.float32).reshape(_C_OUT, 1)
    b_col = beta.astype(jnp.float32).reshape(_C_OUT, 1)

    in_map = lambda ph, st: (st * (1 - ph) + (steps - 1) * ph, 0, 0)
    out_map = lambda ph, st: (st * ph, 0, 0)

    out = pl.pallas_call(
        functools.partial(_fused_kernel, count=float(n * hw), bb=bb,
                          steps=steps),
        out_shape=jax.ShapeDtypeStruct((n, _C_OUT, hw), jnp.float32),
        grid=(2, steps),
        in_specs=[
            pl.BlockSpec((bb, _C_IN, hw), in_map),
            pl.BlockSpec((bb, _C_IN, hw), in_map),
            pl.BlockSpec((_C_OUT, _C_IN), lambda ph, st: (0, 0)),
            pl.BlockSpec((_C_OUT, 1), lambda ph, st: (0, 0)),
            pl.BlockSpec((_C_OUT, 1), lambda ph, st: (0, 0)),
        ],
        out_specs=pl.BlockSpec((bb, _C_OUT, hw), out_map),
        scratch_shapes=[
            pltpu.VMEM((n, _C_IN, hw), jnp.bfloat16),
            pltpu.VMEM((_C_IN, 1), jnp.float32),
            pltpu.VMEM((_C_IN, _C_IN), jnp.float32),
            pltpu.VMEM((_C_OUT, _C_IN), jnp.bfloat16),
            pltpu.VMEM((_C_OUT, 1), jnp.float32),
        ],
        compiler_params=pltpu.CompilerParams(
            dimension_semantics=("arbitrary", "arbitrary")),
    )(x71_r, x57_r, w_mat, g_col, b_col)

    return out.reshape(n, _C_OUT, h, w)


# speedup vs baseline: 1.0363x; 1.0325x over previous
import functools

import jax
import jax.numpy as jnp
from jax.experimental import pallas as pl
from jax.experimental.pallas import tpu as pltpu

_C_IN = 32
_C_OUT = 192
_BN_EPS = 1e-5
_BLOCK_BATCH = 4


def _fused_kernel(x71_ref, x57_ref, w_ref, gamma_ref, beta_ref, o_ref,
                  xs_scr, s_scr, g_scr, wf_scr, bf_scr, *, count, bb, steps):
    phase = pl.program_id(0)
    step = pl.program_id(1)

    @pl.when(jnp.logical_and(phase == 0, step == 0))
    def _init():
        s_scr[...] = jnp.zeros_like(s_scr)
        g_scr[...] = jnp.zeros_like(g_scr)

    @pl.when(phase == 0)
    def _stats():
        x = x71_ref[...] + x57_ref[...]
        xs_scr[pl.ds(step * bb, bb)] = x.astype(jnp.bfloat16)
        s_scr[...] += jnp.sum(x, axis=(0, 2))[:, None]
        g = jnp.zeros((_C_IN, _C_IN), jnp.float32)
        for b in range(bb):
            g = g + jax.lax.dot_general(
                x[b], x[b], (((1,), (1,)), ((), ())),
                preferred_element_type=jnp.float32)
        g_scr[...] += g

    @pl.when(jnp.logical_and(phase == 1, step == 0))
    def _fold():
        inv = 1.0 / count
        mean_x = s_scr[...] * inv
        exx = g_scr[...] * inv
        w = w_ref[...]
        mean_y = jnp.dot(w, mean_x, preferred_element_type=jnp.float32)
        e_y2 = jnp.sum(jnp.dot(w, exx, preferred_element_type=jnp.float32) * w,
                       axis=1, keepdims=True)
        var_y = jnp.maximum(e_y2 - mean_y * mean_y, 0.0)
        scale = gamma_ref[...] * jax.lax.rsqrt(var_y + _BN_EPS)
        wf_scr[...] = (w * scale).astype(jnp.bfloat16)
        bf_scr[...] = beta_ref[...] - mean_y * scale

    @pl.when(phase == 1)
    def _conv():
        wf = wf_scr[...]
        bias = bf_scr[...]
        xsb = xs_scr[pl.ds(step * bb, bb)]
        for b in range(bb):
            y = jnp.dot(wf, xsb[b],
                        preferred_element_type=jnp.float32)
            o_ref[b] = y + bias


def kernel(x71, x57, weight, gamma, beta):
    n, c, h, w = x71.shape
    assert c == _C_IN and x57.shape == x71.shape
    hw = h * w
    bb = _BLOCK_BATCH if n % _BLOCK_BATCH == 0 else 1
    steps = n // bb

    x71_r = x71.reshape(n, _C_IN, hw)
    x57_r = x57.reshape(n, _C_IN, hw)
    w_mat = weight.astype(jnp.float32).reshape(_C_OUT, _C_IN)
    g_col = gamma.astype(jnp.float32).reshape(_C_OUT, 1)
    b_col = beta.astype(jnp.float32).reshape(_C_OUT, 1)

    in_map = lambda ph, st: (st * (1 - ph) + (steps - 1) * ph, 0, 0)
    out_map = lambda ph, st: (st * ph, 0, 0)

    out = pl.pallas_call(
        functools.partial(_fused_kernel, count=float(n * hw), bb=bb,
                          steps=steps),
        out_shape=jax.ShapeDtypeStruct((n, _C_OUT, hw), jnp.float32),
        grid=(2, steps),
        in_specs=[
            pl.BlockSpec((bb, _C_IN, hw), in_map),
            pl.BlockSpec((bb, _C_IN, hw), in_map),
            pl.BlockSpec((_C_OUT, _C_IN), lambda ph, st: (0, 0)),
            pl.BlockSpec((_C_OUT, 1), lambda ph, st: (0, 0)),
            pl.BlockSpec((_C_OUT, 1), lambda ph, st: (0, 0)),
        ],
        out_specs=pl.BlockSpec((bb, _C_OUT, hw), out_map),
        scratch_shapes=[
            pltpu.VMEM((n, _C_IN, hw), jnp.bfloat16),
            pltpu.VMEM((_C_IN, 1), jnp.float32),
            pltpu.VMEM((_C_IN, _C_IN), jnp.float32),
            pltpu.VMEM((_C_OUT, _C_IN), jnp.bfloat16),
            pltpu.VMEM((_C_OUT, 1), jnp.float32),
        ],
        compiler_params=pltpu.CompilerParams(
            dimension_semantics=("arbitrary", "arbitrary")),
    )(x71_r, x57_r, w_mat, g_col, b_col)

    return out.reshape(n, _C_OUT, h, w)
